# baseline (speedup 1.0000x reference)
import jax
import jax.numpy as jnp
from jax.experimental import pallas as pl
from jax.experimental.pallas import tpu as pltpu

_MARGIN = 1.0
_H = 1024
_MH = 512


def _augment_a(e, sq_col, ones_col):
    return jnp.concatenate([e * -2.0, sq_col, ones_col], axis=1)


def _augment_b(e, sq_col, ones_col):
    return jnp.concatenate([e, ones_col, sq_col], axis=1)


def _tile_sum(aug_a, aug_b, lbl_a, lbl_b):
    q = jax.lax.dot_general(
        aug_a, aug_b,
        dimension_numbers=(((1,), (1,)), ((), ())),
        preferred_element_type=jnp.float32,
        precision=jax.lax.Precision.DEFAULT,
    )
    d2 = jnp.maximum(q, 0.0)
    same = jnp.transpose(lbl_a) == lbl_b
    neg_vals = jnp.maximum(_MARGIN - jnp.sqrt(d2), 0.0)
    return jnp.sum(jnp.where(same, d2, neg_vals * neg_vals))


def _body(f1_ref, f2_ref, lbl_ref, o1_ref, t1_ref, o2_ref, t2_ref,
          pair_ref, sse1_ref, sse2_ref):
    p = pl.program_id(0)

    @pl.when(p == 0)
    def _tiles():
        e1 = f1_ref[...]
        e2 = f2_ref[...]
        ones_col = jnp.ones((_H, 1), dtype=jnp.float32)
        sq1 = jnp.sum(e1 * e1, axis=1, keepdims=True)
        sq2 = jnp.sum(e2 * e2, axis=1, keepdims=True)
        a1 = _augment_a(e1, sq1, ones_col)
        b1 = _augment_b(e1, sq1, ones_col)
        a2 = _augment_a(e2, sq2, ones_col)
        b2 = _augment_b(e2, sq2, ones_col)
        l0 = lbl_ref[0:1, :]
        l1 = lbl_ref[1:2, :]
        t00 = _tile_sum(a1, b1, l0, l0)
        t01 = _tile_sum(a1, b2, l0, l1)
        t11 = _tile_sum(a2, b2, l1, l1)
        pair_ref[...] = (0.5 * t00 + t01 + 0.5 * t11)[None, None]

    r1 = o1_ref[...] - t1_ref[...]
    r2 = o2_ref[...] - t2_ref[...]
    s1 = jnp.sum(r1 * r1)[None, None]
    s2 = jnp.sum(r2 * r2)[None, None]

    @pl.when(p == 0)
    def _mse0():
        sse1_ref[...] = s1
        sse2_ref[...] = s2

    @pl.when(p != 0)
    def _mse1():
        sse1_ref[...] += s1
        sse2_ref[...] += s2


def kernel(feature1, feature2, output1, output2, target1, target2, label):
    B, D = output1.shape
    scalar = jax.ShapeDtypeStruct((1, 1), jnp.float32)

    full = lambda shape: pl.BlockSpec(shape, lambda p: (0, 0))
    half = pl.BlockSpec((_MH, D), lambda p: (p, 0))
    pair, sse1, sse2 = pl.pallas_call(
        _body,
        grid=(2,),
        in_specs=[
            full((_H, D)),
            full((_H, D)),
            full((2, _H)),
            half,
            half,
            half,
            half,
        ],
        out_specs=[full((1, 1)), full((1, 1)), full((1, 1))],
        out_shape=[scalar, scalar, scalar],
    )(feature1, feature2, label, output1, target1, output2, target2)

    n = 2 * _H
    n_pairs = jnp.float32(n * (n - 1) / 2)
    denom = jnp.float32(B * D)
    loss1 = sse1[0, 0] / denom
    loss2 = sse2[0, 0] / denom
    loss_mean = pair[0, 0] / n_pairs
    losses = loss_mean + (loss1 + loss2) / 2.0
    return (losses, loss1, loss2, loss_mean)


# speedup vs baseline: 1.0082x; 1.0082x over previous
import jax
import jax.numpy as jnp
from jax.experimental import pallas as pl
from jax.experimental.pallas import tpu as pltpu

_MARGIN = 1.0
_H = 1024


def _rowsq(e):
    return jnp.sum(e * e, axis=1, keepdims=True)


def _tile_sum(e_a, e_b, sq_a_col, sq_b_row, lbl_a, lbl_b):
    dot = jax.lax.dot_general(
        e_a, e_b,
        dimension_numbers=(((1,), (1,)), ((), ())),
        preferred_element_type=jnp.float32,
        precision=jax.lax.Precision.DEFAULT,
    )
    d2 = jnp.maximum(sq_a_col + sq_b_row - 2.0 * dot, 0.0)
    same = jnp.transpose(lbl_a) == lbl_b
    neg_vals = jnp.maximum(_MARGIN - jnp.sqrt(d2), 0.0)
    return jnp.sum(jnp.where(same, d2, neg_vals * neg_vals))


def _body(f1_ref, f2_ref, lbl_ref, o1_ref, t1_ref, o2_ref, t2_ref,
          pair_ref, sse1_ref, sse2_ref, sq1_ref, sq2_ref):
    e1 = f1_ref[...]
    e2 = f2_ref[...]
    sq1_ref[...] = jnp.transpose(_rowsq(e1))
    sq2_ref[...] = jnp.transpose(_rowsq(e2))
    sq1 = sq1_ref[...]
    sq2 = sq2_ref[...]
    l0 = lbl_ref[0:1, :]
    l1 = lbl_ref[1:2, :]
    t00 = _tile_sum(e1, e1, jnp.transpose(sq1), sq1, l0, l0)
    t01 = _tile_sum(e1, e2, jnp.transpose(sq1), sq2, l0, l1)
    t11 = _tile_sum(e2, e2, jnp.transpose(sq2), sq2, l1, l1)
    pair_ref[...] = (0.5 * t00 + t01 + 0.5 * t11)[None, None]
    r1 = o1_ref[...] - t1_ref[...]
    sse1_ref[...] = jnp.sum(r1 * r1)[None, None]
    r2 = o2_ref[...] - t2_ref[...]
    sse2_ref[...] = jnp.sum(r2 * r2)[None, None]


def kernel(feature1, feature2, output1, output2, target1, target2, label):
    B, D = output1.shape
    scalar = jax.ShapeDtypeStruct((1, 1), jnp.float32)

    full = lambda shape: pl.BlockSpec(shape, lambda p: (0, 0))
    pair, sse1, sse2 = pl.pallas_call(
        _body,
        grid=(1,),
        in_specs=[
            full((_H, D)),
            full((_H, D)),
            full((2, _H)),
            full((_H, D)),
            full((_H, D)),
            full((_H, D)),
            full((_H, D)),
        ],
        out_specs=[full((1, 1)), full((1, 1)), full((1, 1))],
        out_shape=[scalar, scalar, scalar],
        scratch_shapes=[pltpu.VMEM((1, _H), jnp.float32),
                        pltpu.VMEM((1, _H), jnp.float32)],
    )(feature1, feature2, label, output1, target1, output2, target2)

    n = 2 * _H
    n_pairs = jnp.float32(n * (n - 1) / 2)
    denom = jnp.float32(B * D)
    loss1 = sse1[0, 0] / denom
    loss2 = sse2[0, 0] / denom
    loss_mean = pair[0, 0] / n_pairs
    losses = loss_mean + (loss1 + loss2) / 2.0
    return (losses, loss1, loss2, loss_mean)
